# baseline (speedup 1.0000x reference)
import functools

import jax
import jax.numpy as jnp
from jax import lax
from jax.experimental import pallas as pl
from jax.experimental.pallas import tpu as pltpu
from jax.experimental.pallas import tpu_sc as plsc

_NC = 2
_NS = 16
_NW = _NC * _NS
_L = 16

_B = 16
_CIN = 10
_COUT = 8
_PIX = 512 * 512

_P = 8192
_CPB = _PIX // _P
_NCHUNK = _B * _CPB
_CPW = _NCHUNK // _NW


def _mag(g8, g9):
    s = g8 * g8 + g9 * g9
    si = lax.bitcast_convert_type(s, jnp.int32)
    yi = jnp.int32(0x5F3759DF) - lax.shift_right_arithmetic(si, jnp.int32(1))
    y = lax.bitcast_convert_type(yi, jnp.float32)
    hs = 0.5 * s
    for _ in range(3):
        y = y * (1.5 - hs * y * y)
    return s * y


def _sc_body(x_hbm, o_hbm, *bufs):
    wid = lax.axis_index("s") * _NC + lax.axis_index("c")

    def chunk_body(t, carry):
        g = wid * _CPW + t
        b = g // _CPB
        off = (g % _CPB) * _P
        for c in range(_CIN):
            pltpu.sync_copy(x_hbm.at[b, c, pl.ds(off, _P)], bufs[c])

        def px_body(i, carry2):
            sl = pl.ds(pl.multiple_of(i * _L, _L), _L)
            vals = [bufs[c][sl] for c in range(_CIN)]
            mag = _mag(vals[8], vals[9])
            best = vals[0]
            bi = jnp.zeros((_L,), jnp.int32)
            for c in range(1, _COUT):
                gt = vals[c] > best
                best = jnp.where(gt, vals[c], best)
                bi = jnp.where(gt, jnp.int32(c), bi)
            zero = jnp.zeros((_L,), jnp.float32)
            for c in range(_COUT):
                bufs[c][sl] = jnp.where(bi == jnp.int32(c), mag, zero)
            return carry2

        lax.fori_loop(0, _P // _L, px_body, 0, unroll=2)

        for c in range(_COUT):
            pltpu.sync_copy(bufs[c], o_hbm.at[b, c, pl.ds(off, _P)])
        return carry

    lax.fori_loop(0, _CPW, chunk_body, 0)


_sc_kernel = pl.kernel(
    _sc_body,
    out_type=jax.ShapeDtypeStruct((_B, _COUT, _PIX), jnp.float32),
    mesh=plsc.VectorSubcoreMesh(core_axis_name="c", subcore_axis_name="s"),
    scratch_types=[pltpu.VMEM((_P,), jnp.float32) for _ in range(_CIN)],
)


def kernel(x):
    B, C, H, W = x.shape
    xr = x.reshape(B, C, H * W)
    out = _sc_kernel(xr)
    return out.reshape(B, _COUT, H, W)


# speedup vs baseline: 1.4824x; 1.4824x over previous
import functools

import jax
import jax.numpy as jnp
from jax import lax
from jax.experimental import pallas as pl
from jax.experimental.pallas import tpu as pltpu
from jax.experimental.pallas import tpu_sc as plsc

_NC = 2
_NS = 16
_NW = _NC * _NS
_L = 16

_B = 16
_CIN = 10
_COUT = 8
_PIX = 512 * 512

_P = 2048
_CPB = _PIX // _P
_NCHUNK = _B * _CPB
_CPW = _NCHUNK // _NW
_NBUF = 2


def _mag(g8, g9):
    s = g8 * g8 + g9 * g9
    si = lax.bitcast_convert_type(s, jnp.int32)
    yi = jnp.int32(0x5F3759DF) - lax.shift_right_arithmetic(si, jnp.int32(1))
    y = lax.bitcast_convert_type(yi, jnp.float32)
    hs = 0.5 * s
    for _ in range(2):
        y = y * (1.5 - hs * y * y)
    return s * y


def _sc_body(x_hbm, o_hbm, ibufs, obufs, isems, osems):
    wid = lax.axis_index("s") * _NC + lax.axis_index("c")
    base = wid * _CPW

    def start_in(slot, chunk):
        b = chunk // _CPB
        off = (chunk % _CPB) * _P
        for c in range(_CIN):
            pltpu.async_copy(x_hbm.at[b, c, pl.ds(off, _P)], ibufs[slot][c],
                             isems[slot])

    def wait_in(slot):
        for c in range(_CIN):
            pltpu.make_async_copy(x_hbm.at[0, 0, pl.ds(0, _P)], ibufs[slot][c],
                                  isems[slot]).wait()

    def start_out(slot, chunk):
        b = chunk // _CPB
        off = (chunk % _CPB) * _P
        for c in range(_COUT):
            pltpu.async_copy(obufs[slot][c], o_hbm.at[b, c, pl.ds(off, _P)],
                             osems[slot])

    def wait_out(slot):
        for c in range(_COUT):
            pltpu.make_async_copy(obufs[slot][c], o_hbm.at[0, 0, pl.ds(0, _P)],
                                  osems[slot]).wait()

    def compute(slot):
        ib = ibufs[slot]
        ob = obufs[slot]

        def px_body(i, carry):
            sl = pl.ds(pl.multiple_of(i * _L, _L), _L)
            vals = [ib[c][sl] for c in range(_CIN)]
            mag = _mag(vals[8], vals[9])
            best = vals[0]
            bi = jnp.zeros((_L,), jnp.int32)
            for c in range(1, _COUT):
                gt = vals[c] > best
                best = jnp.where(gt, vals[c], best)
                bi = jnp.where(gt, jnp.int32(c), bi)
            zero = jnp.zeros((_L,), jnp.float32)
            for c in range(_COUT):
                ob[c][sl] = jnp.where(bi == jnp.int32(c), mag, zero)
            return carry

        lax.fori_loop(0, _P // _L, px_body, 0, unroll=4)

    for s in range(_NBUF):
        start_in(s, base + s)

    def step(t, carry):
        for s in range(_NBUF):
            chunk = base + t + s

            @pl.when(t > 0)
            def _():
                wait_out(s)

            wait_in(s)
            compute(s)
            start_out(s, chunk)

            @pl.when(t + _NBUF < _CPW)
            def _():
                start_in(s, chunk + _NBUF)

        return carry

    lax.fori_loop(0, _CPW // _NBUF, lambda t, c: step(t * _NBUF, c), 0)
    for s in range(_NBUF):
        wait_out(s)


_sc_kernel = pl.kernel(
    _sc_body,
    out_type=jax.ShapeDtypeStruct((_B, _COUT, _PIX), jnp.float32),
    mesh=plsc.VectorSubcoreMesh(core_axis_name="c", subcore_axis_name="s"),
    scratch_types=[
        [[pltpu.VMEM((_P,), jnp.float32) for _ in range(_CIN)] for _ in range(_NBUF)],
        [[pltpu.VMEM((_P,), jnp.float32) for _ in range(_COUT)] for _ in range(_NBUF)],
        [pltpu.SemaphoreType.DMA for _ in range(_NBUF)],
        [pltpu.SemaphoreType.DMA for _ in range(_NBUF)],
    ],
)


def kernel(x):
    B, C, H, W = x.shape
    xr = x.reshape(B, C, H * W)
    out = _sc_kernel(xr)
    return out.reshape(B, _COUT, H, W)


# speedup vs baseline: 4.0586x; 2.7377x over previous
import functools

import jax
import jax.numpy as jnp
from jax import lax
from jax.experimental import pallas as pl
from jax.experimental.pallas import tpu as pltpu
from jax.experimental.pallas import tpu_sc as plsc

_NC = 2
_NS = 16
_NW = _NC * _NS
_L = 16

_B = 16
_CIN = 10
_COUT = 8
_H = 512
_W = 512

_R = 8
_CPP = _H // _R
_NCHUNK = _B * _CPP
_CPW = _NCHUNK // _NW
_NBUF = 2
_GRP = _R * _W // _L


def _mag(g8, g9):
    s = g8 * g8 + g9 * g9
    si = lax.bitcast_convert_type(s, jnp.int32)
    yi = jnp.int32(0x5F3759DF) - lax.shift_right_arithmetic(si, jnp.int32(1))
    y = lax.bitcast_convert_type(yi, jnp.float32)
    hs = 0.5 * s
    for _ in range(2):
        y = y * (1.5 - hs * y * y)
    return s * y


def _sc_body(x_hbm, o_hbm, bufs, isems, osems):
    wid = lax.axis_index("s") * _NC + lax.axis_index("c")
    base = wid * _CPW

    def start_in(slot, chunk):
        b = chunk // _CPP
        r0 = (chunk % _CPP) * _R
        for c in range(_CIN):
            pltpu.async_copy(x_hbm.at[b, c, pl.ds(r0, _R), :], bufs[slot][c],
                             isems[slot])

    def wait_in(slot):
        for c in range(_CIN):
            pltpu.make_async_copy(x_hbm.at[0, 0, pl.ds(0, _R), :], bufs[slot][c],
                                  isems[slot]).wait()

    def start_out(slot, chunk):
        b = chunk // _CPP
        r0 = (chunk % _CPP) * _R
        for c in range(_COUT):
            pltpu.async_copy(bufs[slot][c], o_hbm.at[b, c, pl.ds(r0, _R), :],
                             osems[slot])

    def wait_out(slot):
        for c in range(_COUT):
            pltpu.make_async_copy(bufs[slot][c], o_hbm.at[0, 0, pl.ds(0, _R), :],
                                  osems[slot]).wait()

    def compute(slot):
        ib = bufs[slot]

        def px_body(j, carry):
            col = pl.ds(pl.multiple_of(j * _L, _L), _L)
            for r in range(_R):
                vals = [ib[c][r, col] for c in range(_CIN)]
                mag = _mag(vals[8], vals[9])
                best = vals[0]
                bi = jnp.zeros((_L,), jnp.int32)
                for c in range(1, _COUT):
                    gt = vals[c] > best
                    best = jnp.where(gt, vals[c], best)
                    bi = jnp.where(gt, jnp.int32(c), bi)
                zero = jnp.zeros((_L,), jnp.float32)
                for c in range(_COUT):
                    ib[c][r, col] = jnp.where(bi == jnp.int32(c), mag, zero)
            return carry

        lax.fori_loop(0, _W // _L, px_body, 0)

    for s in range(_NBUF):
        start_in(s, base + s)

    def step(t, carry):
        for s in range(_NBUF):
            chunk = base + t + s
            wait_in(s)
            compute(s)
            start_out(s, chunk)

            @pl.when(t + s + _NBUF < _CPW)
            def _():
                wait_out(s)
                start_in(s, chunk + _NBUF)

        return carry

    lax.fori_loop(0, _CPW // _NBUF, lambda t, c: step(t * _NBUF, c), 0)
    for s in range(_NBUF):
        wait_out(s)


_sc_kernel = pl.kernel(
    _sc_body,
    out_type=jax.ShapeDtypeStruct((_B, _COUT, _H, _W), jnp.float32),
    mesh=plsc.VectorSubcoreMesh(core_axis_name="c", subcore_axis_name="s"),
    compiler_params=pltpu.CompilerParams(use_tc_tiling_on_sc=True),
    scratch_types=[
        [[pltpu.VMEM((_R, _W), jnp.float32) for _ in range(_CIN)]
         for _ in range(_NBUF)],
        [pltpu.SemaphoreType.DMA for _ in range(_NBUF)],
        [pltpu.SemaphoreType.DMA for _ in range(_NBUF)],
    ],
)


def kernel(x):
    return _sc_kernel(x)
